# baseline (speedup 1.0000x reference)
import functools

import jax
import jax.numpy as jnp
from jax import lax
from jax.experimental import pallas as pl
from jax.experimental.pallas import tpu as pltpu
from jax.experimental.pallas import tpu_sc as plsc

D = 2048
I = 1408
E = 8
T = 4096
A = 2 * T
BM = 128
CAP = A + E * BM
NB = CAP // BM
NW = 32
CH = 16


def _router_body(x_ref, gw_ref, ids_ref, w_ref):
    l = jnp.dot(x_ref[...], gw_ref[...], preferred_element_type=jnp.float32)
    idx8 = lax.broadcasted_iota(jnp.int32, l.shape, 1)
    m0 = jnp.max(l, axis=-1, keepdims=True)
    e0 = jnp.min(jnp.where(l == m0, idx8, E), axis=-1, keepdims=True)
    lm = jnp.where(idx8 == e0, -jnp.inf, l)
    m1 = jnp.max(lm, axis=-1, keepdims=True)
    e1 = jnp.min(jnp.where(lm == m1, idx8, E), axis=-1, keepdims=True)
    p1 = jnp.exp(m1 - m0)
    w0 = 1.0 / (1.0 + p1)
    ids_ref[...] = jnp.concatenate([e0, e1], axis=1)
    w_ref[...] = jnp.concatenate([w0, 1.0 - w0], axis=1)


def _router(x, gw_t):
    bt = 1024
    return pl.pallas_call(
        _router_body,
        grid=(T // bt,),
        in_specs=[
            pl.BlockSpec((bt, D), lambda i: (i, 0)),
            pl.BlockSpec((D, E), lambda i: (0, 0)),
        ],
        out_specs=[
            pl.BlockSpec((bt, 2), lambda i: (i, 0)),
            pl.BlockSpec((bt, 2), lambda i: (i, 0)),
        ],
        out_shape=[
            jax.ShapeDtypeStruct((T, 2), jnp.int32),
            jax.ShapeDtypeStruct((T, 2), jnp.float32),
        ],
    )(x, gw_t)


def _dispatch_body(ef_ref, dest_ref, meta_ref):
    ef = ef_ref[...]
    r128 = lax.broadcasted_iota(jnp.int32, (128, 128), 0)
    c128 = lax.broadcasted_iota(jnp.int32, (128, 128), 1)
    upper = (r128 <= c128).astype(jnp.float32)
    r64 = lax.broadcasted_iota(jnp.int32, (64, 64), 0)
    c64 = lax.broadcasted_iota(jnp.int32, (64, 64), 1)
    lower = (c64 < r64).astype(jnp.float32)

    masks, incls, cnts = [], [], []
    for e in range(E):
        a = (ef == e).astype(jnp.float32)
        incl_row = jnp.dot(a, upper, preferred_element_type=jnp.float32)
        row_sum = jnp.sum(a, axis=1, keepdims=True)
        prev_rows = jnp.dot(lower, row_sum, preferred_element_type=jnp.float32)
        masks.append(a)
        incls.append(incl_row + prev_rows)
        cnts.append(jnp.sum(a).astype(jnp.int32))

    off = jnp.int32(0)
    ends = []
    dest = jnp.zeros((64, 128), jnp.int32)
    for e in range(E):
        dest = dest + masks[e].astype(jnp.int32) * (
            off + incls[e].astype(jnp.int32) - 1)
        off = off + ((cnts[e] + BM - 1) // BM) * BM
        ends.append(off)
    nreal = off // BM

    bstart = lax.broadcasted_iota(jnp.int32, (8, 128), 1) * BM
    be = jnp.zeros((8, 128), jnp.int32)
    for e in range(E):
        be = be + (bstart >= ends[e]).astype(jnp.int32)
    be = jnp.minimum(be, E - 1)
    rowi = lax.broadcasted_iota(jnp.int32, (8, 128), 0)
    dest_ref[...] = dest
    meta_ref[...] = jnp.where(rowi == 1, nreal, be)


def _dispatch(efr):
    return pl.pallas_call(
        _dispatch_body,
        out_shape=[
            jax.ShapeDtypeStruct((64, 128), jnp.int32),
            jax.ShapeDtypeStruct((8, 128), jnp.int32),
        ],
    )(efr)


NCH = A // (NW * CH)


def _sc_scatter(x, tok2d, dest2d):
    mesh = plsc.VectorSubcoreMesh(
        core_axis_name="c", subcore_axis_name="s", num_cores=2,
        num_subcores=16)

    @functools.partial(
        pl.kernel,
        out_type=jax.ShapeDtypeStruct((CAP, D), jnp.float32),
        mesh=mesh,
        scratch_types=[
            pltpu.VMEM((NCH, CH), jnp.int32),
            pltpu.VMEM((NCH, CH), jnp.int32),
            pltpu.VMEM((CH, D), jnp.float32),
            pltpu.VMEM((CH, D), jnp.float32),
            pltpu.SemaphoreType.DMA,
            pltpu.SemaphoreType.DMA,
            pltpu.SemaphoreType.DMA,
            pltpu.SemaphoreType.DMA,
        ],
    )
    def k(x_hbm, tok_hbm, dest_hbm, xs_hbm, tok_v, dest_v,
          rows_a, rows_b, gs_a, gs_b, ss_a, ss_b):
        wid = lax.axis_index("s") * 2 + lax.axis_index("c")
        pltpu.sync_copy(tok_hbm.at[pl.ds(wid * NCH, NCH)], tok_v)
        pltpu.sync_copy(dest_hbm.at[pl.ds(wid * NCH, NCH)], dest_v)
        rows = (rows_a, rows_b)
        gsem = (gs_a, gs_b)
        ssem = (ss_a, ss_b)
        gd = [None, None]
        sd = [None, None]
        gd[0] = pltpu.async_copy(x_hbm.at[tok_v.at[0]], rows[0], gsem[0])
        for c in range(NCH):
            b = c % 2
            if sd[1 - b] is not None:
                sd[1 - b].wait()
            if c + 1 < NCH:
                gd[1 - b] = pltpu.async_copy(
                    x_hbm.at[tok_v.at[c + 1]], rows[1 - b], gsem[1 - b])
            gd[b].wait()
            sd[b] = pltpu.async_copy(rows[b], xs_hbm.at[dest_v.at[c]], ssem[b])
        sd[(NCH - 1) % 2].wait()

    return k(x, tok2d, dest2d)


def _moe_body(pref_ref, xs_ref, wg_ref, wu_ref, wd_ref, ys_ref):
    m = pl.program_id(0)

    @pl.when(m < pref_ref[NB])
    def _():
        xb = xs_ref[...].astype(jnp.bfloat16)
        g = jnp.dot(xb, wg_ref[0], preferred_element_type=jnp.float32)
        u = jnp.dot(xb, wu_ref[0], preferred_element_type=jnp.float32)
        act = (g / (1.0 + jnp.exp(-g))) * u
        ys_ref[...] = jnp.dot(act.astype(jnp.bfloat16), wd_ref[0],
                              preferred_element_type=jnp.float32)


def _moe(pref, xs, wgb, wub, wdb):
    grid_spec = pltpu.PrefetchScalarGridSpec(
        num_scalar_prefetch=1,
        grid=(NB,),
        in_specs=[
            pl.BlockSpec((BM, D), lambda m, p: (m, 0)),
            pl.BlockSpec((1, D, I), lambda m, p: (p[m], 0, 0)),
            pl.BlockSpec((1, D, I), lambda m, p: (p[m], 0, 0)),
            pl.BlockSpec((1, I, D), lambda m, p: (p[m], 0, 0)),
        ],
        out_specs=pl.BlockSpec((BM, D), lambda m, p: (m, 0)),
    )
    return pl.pallas_call(
        _moe_body,
        grid_spec=grid_spec,
        out_shape=jax.ShapeDtypeStruct((CAP, D), jnp.float32),
    )(pref, xs, wgb, wub, wdb)


def _sc_gather(ys, dest2d):
    mesh = plsc.VectorSubcoreMesh(
        core_axis_name="c", subcore_axis_name="s", num_cores=2,
        num_subcores=16)

    @functools.partial(
        pl.kernel,
        out_type=jax.ShapeDtypeStruct((A, D), jnp.float32),
        mesh=mesh,
        scratch_types=[
            pltpu.VMEM((NCH, CH), jnp.int32),
            pltpu.VMEM((CH, D), jnp.float32),
            pltpu.VMEM((CH, D), jnp.float32),
            pltpu.SemaphoreType.DMA,
            pltpu.SemaphoreType.DMA,
            pltpu.SemaphoreType.DMA,
            pltpu.SemaphoreType.DMA,
        ],
    )
    def k(ys_hbm, dest_hbm, yp_hbm, dest_v, rows_a, rows_b,
          gs_a, gs_b, os_a, os_b):
        wid = lax.axis_index("s") * 2 + lax.axis_index("c")
        pltpu.sync_copy(dest_hbm.at[pl.ds(wid * NCH, NCH)], dest_v)
        rows = (rows_a, rows_b)
        gsem = (gs_a, gs_b)
        osem = (os_a, os_b)
        gd = [None, None]
        od = [None, None]
        gd[0] = pltpu.async_copy(ys_hbm.at[dest_v.at[0]], rows[0], gsem[0])
        for c in range(NCH):
            b = c % 2
            if od[1 - b] is not None:
                od[1 - b].wait()
            if c + 1 < NCH:
                gd[1 - b] = pltpu.async_copy(
                    ys_hbm.at[dest_v.at[c + 1]], rows[1 - b], gsem[1 - b])
            gd[b].wait()
            base = wid * (A // NW) + c * CH
            od[b] = pltpu.async_copy(rows[b], yp_hbm.at[pl.ds(base, CH)],
                                     osem[b])
        od[(NCH - 1) % 2].wait()

    return k(ys, dest2d)


def _combine_body(yp_ref, w_ref, o_ref):
    w = w_ref[...]
    o_ref[...] = (yp_ref[:, 0, :] * w[:, 0:1] +
                  yp_ref[:, 1, :] * w[:, 1:2])


def _combine(yp3, w):
    bt = 512
    return pl.pallas_call(
        _combine_body,
        grid=(T // bt,),
        in_specs=[
            pl.BlockSpec((bt, 2, D), lambda i: (i, 0, 0)),
            pl.BlockSpec((bt, 2), lambda i: (i, 0)),
        ],
        out_specs=pl.BlockSpec((bt, D), lambda i: (i, 0)),
        out_shape=jax.ShapeDtypeStruct((T, D), jnp.float32),
    )(yp3, w)


def kernel(hidden_states, gate_weight, gate_up_weights, down_weights):
    x = hidden_states.reshape(-1, D)
    gw_t = gate_weight.T
    wgb = gate_up_weights[:, :, :I].astype(jnp.bfloat16)
    wub = gate_up_weights[:, :, I:].astype(jnp.bfloat16)
    wdb = down_weights.astype(jnp.bfloat16)

    ids, w = _router(x, gw_t)
    dest2, meta = _dispatch(ids.reshape(64, 128))
    dest2d = dest2.reshape(A // CH, CH)
    pref = jnp.concatenate([meta[0, :NB], meta[1, :1]]).astype(jnp.int32)
    tok2d = (jnp.arange(A, dtype=jnp.int32) // 2).reshape(A // CH, CH)

    xs = _sc_scatter(x, tok2d, dest2d)
    ys = _moe(pref, xs, wgb, wub, wdb)
    yp = _sc_gather(ys, dest2d)
    out = _combine(yp.reshape(T, 2, D), w)
    return out.reshape(hidden_states.shape)


# speedup vs baseline: 1.0246x; 1.0246x over previous
import functools

import jax
import jax.numpy as jnp
from jax import lax
from jax.experimental import pallas as pl
from jax.experimental.pallas import tpu as pltpu
from jax.experimental.pallas import tpu_sc as plsc

D = 2048
I = 1408
E = 8
T = 4096
A = 2 * T
BM = 128
CAP = A + E * BM
NB = CAP // BM
NW = 32
CH = 32
P = D // 2


def _router_body(x_ref, gw_ref, ids_ref, w_ref):
    l = jnp.dot(x_ref[...], gw_ref[...], preferred_element_type=jnp.float32)
    idx8 = lax.broadcasted_iota(jnp.int32, l.shape, 1)
    m0 = jnp.max(l, axis=-1, keepdims=True)
    e0 = jnp.min(jnp.where(l == m0, idx8, E), axis=-1, keepdims=True)
    lm = jnp.where(idx8 == e0, -jnp.inf, l)
    m1 = jnp.max(lm, axis=-1, keepdims=True)
    e1 = jnp.min(jnp.where(lm == m1, idx8, E), axis=-1, keepdims=True)
    p1 = jnp.exp(m1 - m0)
    w0 = 1.0 / (1.0 + p1)
    ids_ref[...] = jnp.concatenate([e0, e1], axis=1)
    w_ref[...] = jnp.concatenate([w0, 1.0 - w0], axis=1)


def _router(x, gw_t):
    bt = 1024
    return pl.pallas_call(
        _router_body,
        grid=(T // bt,),
        in_specs=[
            pl.BlockSpec((bt, D), lambda i: (i, 0)),
            pl.BlockSpec((D, E), lambda i: (0, 0)),
        ],
        out_specs=[
            pl.BlockSpec((bt, 2), lambda i: (i, 0)),
            pl.BlockSpec((bt, 2), lambda i: (i, 0)),
        ],
        out_shape=[
            jax.ShapeDtypeStruct((T, 2), jnp.int32),
            jax.ShapeDtypeStruct((T, 2), jnp.float32),
        ],
    )(x, gw_t)


def _dispatch_body(ef_ref, dest_ref, meta_ref):
    ef = ef_ref[...]
    r128 = lax.broadcasted_iota(jnp.int32, (128, 128), 0)
    c128 = lax.broadcasted_iota(jnp.int32, (128, 128), 1)
    upper = (r128 <= c128).astype(jnp.float32)
    r64 = lax.broadcasted_iota(jnp.int32, (64, 64), 0)
    c64 = lax.broadcasted_iota(jnp.int32, (64, 64), 1)
    lower = (c64 < r64).astype(jnp.float32)

    masks, incls, cnts = [], [], []
    for e in range(E):
        a = (ef == e).astype(jnp.float32)
        incl_row = jnp.dot(a, upper, preferred_element_type=jnp.float32)
        row_sum = jnp.sum(a, axis=1, keepdims=True)
        prev_rows = jnp.dot(lower, row_sum, preferred_element_type=jnp.float32)
        masks.append(a)
        incls.append(incl_row + prev_rows)
        cnts.append(jnp.sum(a).astype(jnp.int32))

    off = jnp.int32(0)
    ends = []
    dest = jnp.zeros((64, 128), jnp.int32)
    for e in range(E):
        dest = dest + masks[e].astype(jnp.int32) * (
            off + incls[e].astype(jnp.int32) - 1)
        off = off + ((cnts[e] + BM - 1) // BM) * BM
        ends.append(off)
    nreal = off // BM

    bstart = lax.broadcasted_iota(jnp.int32, (8, 128), 1) * BM
    be = jnp.zeros((8, 128), jnp.int32)
    for e in range(E):
        be = be + (bstart >= ends[e]).astype(jnp.int32)
    be = jnp.minimum(be, E - 1)
    rowi = lax.broadcasted_iota(jnp.int32, (8, 128), 0)
    dest_ref[...] = dest
    meta_ref[...] = jnp.where(rowi == 1, nreal, be)


def _dispatch(efr):
    return pl.pallas_call(
        _dispatch_body,
        out_shape=[
            jax.ShapeDtypeStruct((64, 128), jnp.int32),
            jax.ShapeDtypeStruct((8, 128), jnp.int32),
        ],
    )(efr)


NCH = A // (NW * CH)


def _sc_scatter(x, tok2d, dest2d):
    mesh = plsc.VectorSubcoreMesh(
        core_axis_name="c", subcore_axis_name="s", num_cores=2,
        num_subcores=16)

    @functools.partial(
        pl.kernel,
        out_type=jax.ShapeDtypeStruct((CAP, P), jnp.int32),
        mesh=mesh,
        scratch_types=[
            pltpu.VMEM((NCH, CH), jnp.int32),
            pltpu.VMEM((NCH, CH), jnp.int32),
            pltpu.VMEM((CH, P), jnp.int32),
            pltpu.VMEM((CH, P), jnp.int32),
            pltpu.SemaphoreType.DMA,
            pltpu.SemaphoreType.DMA,
            pltpu.SemaphoreType.DMA,
            pltpu.SemaphoreType.DMA,
        ],
    )
    def k(x_hbm, tok_hbm, dest_hbm, xs_hbm, tok_v, dest_v,
          rows_a, rows_b, gs_a, gs_b, ss_a, ss_b):
        wid = lax.axis_index("s") * 2 + lax.axis_index("c")
        pltpu.sync_copy(tok_hbm.at[pl.ds(wid * NCH, NCH)], tok_v)
        pltpu.sync_copy(dest_hbm.at[pl.ds(wid * NCH, NCH)], dest_v)
        rows = (rows_a, rows_b)
        gsem = (gs_a, gs_b)
        ssem = (ss_a, ss_b)
        gd = [None, None]
        sd = [None, None]
        gd[0] = pltpu.async_copy(x_hbm.at[tok_v.at[0]], rows[0], gsem[0])
        for c in range(NCH):
            b = c % 2
            if sd[1 - b] is not None:
                sd[1 - b].wait()
            if c + 1 < NCH:
                gd[1 - b] = pltpu.async_copy(
                    x_hbm.at[tok_v.at[c + 1]], rows[1 - b], gsem[1 - b])
            gd[b].wait()
            sd[b] = pltpu.async_copy(rows[b], xs_hbm.at[dest_v.at[c]], ssem[b])
        sd[(NCH - 1) % 2].wait()

    return k(x, tok2d, dest2d)


def _moe_body(pref_ref, xs_ref, wg_ref, wu_ref, wd_ref, ys_ref):
    m = pl.program_id(0)

    @pl.when(m < pref_ref[NB])
    def _():
        x2 = pltpu.bitcast(xs_ref[...], jnp.bfloat16).reshape(BM, 2, P)
        xlo = x2[:, 0, :]
        xhi = x2[:, 1, :]
        g = (jnp.dot(xlo, wg_ref[0, 0], preferred_element_type=jnp.float32) +
             jnp.dot(xhi, wg_ref[0, 1], preferred_element_type=jnp.float32))
        u = (jnp.dot(xlo, wu_ref[0, 0], preferred_element_type=jnp.float32) +
             jnp.dot(xhi, wu_ref[0, 1], preferred_element_type=jnp.float32))
        act = (g / (1.0 + jnp.exp(-g))) * u
        y = jnp.dot(act.astype(jnp.bfloat16), wd_ref[0],
                    preferred_element_type=jnp.float32)
        yb = y.astype(jnp.bfloat16).reshape(2 * BM, P)
        ys_ref[...] = pltpu.bitcast(yb, jnp.int32)


def _moe(pref, xs, wgb, wub, wdb):
    grid_spec = pltpu.PrefetchScalarGridSpec(
        num_scalar_prefetch=1,
        grid=(NB,),
        in_specs=[
            pl.BlockSpec((BM, P), lambda m, p: (m, 0)),
            pl.BlockSpec((1, 2, P, I), lambda m, p: (p[m], 0, 0, 0)),
            pl.BlockSpec((1, 2, P, I), lambda m, p: (p[m], 0, 0, 0)),
            pl.BlockSpec((1, I, D), lambda m, p: (p[m], 0, 0)),
        ],
        out_specs=pl.BlockSpec((BM, P), lambda m, p: (m, 0)),
    )
    return pl.pallas_call(
        _moe_body,
        grid_spec=grid_spec,
        out_shape=jax.ShapeDtypeStruct((CAP, P), jnp.int32),
    )(pref, xs, wgb, wub, wdb)


def _sc_gather(ys, dest2d):
    mesh = plsc.VectorSubcoreMesh(
        core_axis_name="c", subcore_axis_name="s", num_cores=2,
        num_subcores=16)

    @functools.partial(
        pl.kernel,
        out_type=jax.ShapeDtypeStruct((A, P), jnp.int32),
        mesh=mesh,
        scratch_types=[
            pltpu.VMEM((NCH, CH), jnp.int32),
            pltpu.VMEM((CH, P), jnp.int32),
            pltpu.VMEM((CH, P), jnp.int32),
            pltpu.SemaphoreType.DMA,
            pltpu.SemaphoreType.DMA,
            pltpu.SemaphoreType.DMA,
            pltpu.SemaphoreType.DMA,
        ],
    )
    def k(ys_hbm, dest_hbm, yp_hbm, dest_v, rows_a, rows_b,
          gs_a, gs_b, os_a, os_b):
        wid = lax.axis_index("s") * 2 + lax.axis_index("c")
        pltpu.sync_copy(dest_hbm.at[pl.ds(wid * NCH, NCH)], dest_v)
        rows = (rows_a, rows_b)
        gsem = (gs_a, gs_b)
        osem = (os_a, os_b)
        gd = [None, None]
        od = [None, None]
        gd[0] = pltpu.async_copy(ys_hbm.at[dest_v.at[0]], rows[0], gsem[0])
        for c in range(NCH):
            b = c % 2
            if od[1 - b] is not None:
                od[1 - b].wait()
            if c + 1 < NCH:
                gd[1 - b] = pltpu.async_copy(
                    ys_hbm.at[dest_v.at[c + 1]], rows[1 - b], gsem[1 - b])
            gd[b].wait()
            base = wid * (A // NW) + c * CH
            od[b] = pltpu.async_copy(rows[b], yp_hbm.at[pl.ds(base, CH)],
                                     osem[b])
        od[(NCH - 1) % 2].wait()

    return k(ys, dest2d)


def _combine_body(yp_ref, w_ref, o_ref):
    bt = w_ref.shape[0]
    r4 = pltpu.bitcast(yp_ref[...], jnp.bfloat16).reshape(bt, 2, 2, P)
    w = w_ref[...]
    w0 = w[:, 0:1]
    w1 = w[:, 1:2]
    out_lo = r4[:, 0, 0, :] * w0 + r4[:, 1, 0, :] * w1
    out_hi = r4[:, 0, 1, :] * w0 + r4[:, 1, 1, :] * w1
    o_ref[...] = jnp.concatenate([out_lo, out_hi], axis=-1)


def _combine(yp, w):
    bt = 512
    return pl.pallas_call(
        _combine_body,
        grid=(T // bt,),
        in_specs=[
            pl.BlockSpec((2 * bt, P), lambda i: (i, 0)),
            pl.BlockSpec((bt, 2), lambda i: (i, 0)),
        ],
        out_specs=pl.BlockSpec((bt, D), lambda i: (i, 0)),
        out_shape=jax.ShapeDtypeStruct((T, D), jnp.float32),
    )(yp, w)


def kernel(hidden_states, gate_weight, gate_up_weights, down_weights):
    x = hidden_states.reshape(-1, D)
    gw_t = gate_weight.T
    wgb = gate_up_weights[:, :, :I].astype(jnp.bfloat16).reshape(E, 2, P, I)
    wub = gate_up_weights[:, :, I:].astype(jnp.bfloat16).reshape(E, 2, P, I)
    wdb = down_weights.astype(jnp.bfloat16)
    xbf = x.astype(jnp.bfloat16)
    xp = lax.bitcast_convert_type(
        jnp.stack([xbf[:, :P], xbf[:, P:]], axis=-1), jnp.int32)

    ids, w = _router(x, gw_t)
    dest2, meta = _dispatch(ids.reshape(64, 128))
    dest2d = dest2.reshape(A // CH, CH)
    pref = jnp.concatenate([meta[0, :NB], meta[1, :1]]).astype(jnp.int32)
    tok2d = (jnp.arange(A, dtype=jnp.int32) // 2).reshape(A // CH, CH)

    xs = _sc_scatter(xp, tok2d, dest2d)
    ys = _moe(pref, xs, wgb, wub, wdb)
    yp = _sc_gather(ys, dest2d)
    out = _combine(yp, w)
    return out.reshape(hidden_states.shape)


# speedup vs baseline: 1.3123x; 1.2809x over previous
import functools

import jax
import jax.numpy as jnp
from jax import lax
from jax.experimental import pallas as pl
from jax.experimental.pallas import tpu as pltpu
from jax.experimental.pallas import tpu_sc as plsc

D = 2048
I = 1408
E = 8
T = 4096
A = 2 * T
BM = 128
CAP = A + E * BM
NB = CAP // BM
NW = 32
CH = 32
P = D // 2


def _router_body(x_ref, gw_ref, ids_ref, w_ref, xp_ref):
    xb = x_ref[...]
    l = jnp.dot(xb, gw_ref[...], preferred_element_type=jnp.float32)
    idx8 = lax.broadcasted_iota(jnp.int32, l.shape, 1)
    m0 = jnp.max(l, axis=-1, keepdims=True)
    e0 = jnp.min(jnp.where(l == m0, idx8, E), axis=-1, keepdims=True)
    lm = jnp.where(idx8 == e0, -jnp.inf, l)
    m1 = jnp.max(lm, axis=-1, keepdims=True)
    e1 = jnp.min(jnp.where(lm == m1, idx8, E), axis=-1, keepdims=True)
    p1 = jnp.exp(m1 - m0)
    w0 = 1.0 / (1.0 + p1)
    ids_ref[...] = jnp.concatenate([e0, e1], axis=1)
    w_ref[...] = jnp.concatenate([w0, 1.0 - w0], axis=1)
    bt = xb.shape[0]
    xp_ref[...] = pltpu.bitcast(
        xb.astype(jnp.bfloat16).reshape(2 * bt, P), jnp.int32)


def _router(x, gw_t):
    bt = 1024
    return pl.pallas_call(
        _router_body,
        grid=(T // bt,),
        in_specs=[
            pl.BlockSpec((bt, D), lambda i: (i, 0)),
            pl.BlockSpec((D, E), lambda i: (0, 0)),
        ],
        out_specs=[
            pl.BlockSpec((bt, 2), lambda i: (i, 0)),
            pl.BlockSpec((bt, 2), lambda i: (i, 0)),
            pl.BlockSpec((bt, P), lambda i: (i, 0)),
        ],
        out_shape=[
            jax.ShapeDtypeStruct((T, 2), jnp.int32),
            jax.ShapeDtypeStruct((T, 2), jnp.float32),
            jax.ShapeDtypeStruct((T, P), jnp.int32),
        ],
    )(x, gw_t)


def _dispatch_body(ef_ref, dest_ref, meta_ref):
    ef = ef_ref[...]
    r128 = lax.broadcasted_iota(jnp.int32, (128, 128), 0)
    c128 = lax.broadcasted_iota(jnp.int32, (128, 128), 1)
    upper = (r128 <= c128).astype(jnp.float32)
    r64 = lax.broadcasted_iota(jnp.int32, (64, 64), 0)
    c64 = lax.broadcasted_iota(jnp.int32, (64, 64), 1)
    lower = (c64 < r64).astype(jnp.float32)

    masks, incls, cnts = [], [], []
    for e in range(E):
        a = (ef == e).astype(jnp.float32)
        incl_row = jnp.dot(a, upper, preferred_element_type=jnp.float32)
        row_sum = jnp.sum(a, axis=1, keepdims=True)
        prev_rows = jnp.dot(lower, row_sum, preferred_element_type=jnp.float32)
        masks.append(a)
        incls.append(incl_row + prev_rows)
        cnts.append(jnp.sum(a).astype(jnp.int32))

    off = jnp.int32(0)
    ends = []
    dest = jnp.zeros((64, 128), jnp.int32)
    for e in range(E):
        dest = dest + masks[e].astype(jnp.int32) * (
            off + incls[e].astype(jnp.int32) - 1)
        off = off + ((cnts[e] + BM - 1) // BM) * BM
        ends.append(off)
    nreal = off // BM

    bstart = lax.broadcasted_iota(jnp.int32, (8, 128), 1) * BM
    be = jnp.zeros((8, 128), jnp.int32)
    for e in range(E):
        be = be + (bstart >= ends[e]).astype(jnp.int32)
    be = jnp.minimum(be, E - 1)
    rowi = lax.broadcasted_iota(jnp.int32, (8, 128), 0)
    dest_ref[...] = dest
    meta_ref[...] = jnp.where(rowi == 1, nreal, be)


def _dispatch(efr):
    return pl.pallas_call(
        _dispatch_body,
        out_shape=[
            jax.ShapeDtypeStruct((64, 128), jnp.int32),
            jax.ShapeDtypeStruct((8, 128), jnp.int32),
        ],
    )(efr)


NCH = A // (NW * CH)


def _sc_scatter(x, tok2d, dest2d):
    mesh = plsc.VectorSubcoreMesh(
        core_axis_name="c", subcore_axis_name="s", num_cores=2,
        num_subcores=16)

    @functools.partial(
        pl.kernel,
        out_type=jax.ShapeDtypeStruct((CAP, P), jnp.int32),
        mesh=mesh,
        scratch_types=[
            pltpu.VMEM((NCH, CH), jnp.int32),
            pltpu.VMEM((NCH, CH), jnp.int32),
            pltpu.VMEM((CH, P), jnp.int32),
            pltpu.VMEM((CH, P), jnp.int32),
            pltpu.SemaphoreType.DMA,
            pltpu.SemaphoreType.DMA,
            pltpu.SemaphoreType.DMA,
            pltpu.SemaphoreType.DMA,
        ],
    )
    def k(x_hbm, tok_hbm, dest_hbm, xs_hbm, tok_v, dest_v,
          rows_a, rows_b, gs_a, gs_b, ss_a, ss_b):
        wid = lax.axis_index("s") * 2 + lax.axis_index("c")
        pltpu.sync_copy(tok_hbm.at[pl.ds(wid * NCH, NCH)], tok_v)
        pltpu.sync_copy(dest_hbm.at[pl.ds(wid * NCH, NCH)], dest_v)
        rows = (rows_a, rows_b)
        gsem = (gs_a, gs_b)
        ssem = (ss_a, ss_b)
        gd = [None, None]
        sd = [None, None]
        gd[0] = pltpu.async_copy(x_hbm.at[tok_v.at[0]], rows[0], gsem[0])
        for c in range(NCH):
            b = c % 2
            if sd[1 - b] is not None:
                sd[1 - b].wait()
            if c + 1 < NCH:
                gd[1 - b] = pltpu.async_copy(
                    x_hbm.at[tok_v.at[c + 1]], rows[1 - b], gsem[1 - b])
            gd[b].wait()
            sd[b] = pltpu.async_copy(rows[b], xs_hbm.at[dest_v.at[c]], ssem[b])
        sd[(NCH - 1) % 2].wait()

    return k(x, tok2d, dest2d)


_WSPEC = pl.Buffered(buffer_count=2)


def _m1_body(pref_ref, xs_ref, wgu_ref, act_ref, wgu_bf):
    m = pl.program_id(0)
    prev = pref_ref[jnp.maximum(m - 1, 0)]

    @pl.when((m == 0) | (pref_ref[m] != prev))
    def _():
        wgu_bf[...] = wgu_ref[0].astype(jnp.bfloat16)

    @pl.when(m < pref_ref[NB])
    def _():
        x2 = pltpu.bitcast(xs_ref[...], jnp.bfloat16).reshape(BM, 2, P)
        gu = (jnp.dot(x2[:, 0, :], wgu_bf[:P, :],
                      preferred_element_type=jnp.float32) +
              jnp.dot(x2[:, 1, :], wgu_bf[P:, :],
                      preferred_element_type=jnp.float32))
        g = gu[:, :I]
        u = gu[:, I:]
        act_ref[...] = ((g / (1.0 + jnp.exp(-g))) * u).astype(jnp.bfloat16)


def _m2_body(pref_ref, act_ref, wd_ref, ys_ref, wd_bf):
    m = pl.program_id(0)
    prev = pref_ref[jnp.maximum(m - 1, 0)]

    @pl.when((m == 0) | (pref_ref[m] != prev))
    def _():
        wd_bf[...] = wd_ref[0].astype(jnp.bfloat16)

    @pl.when(m < pref_ref[NB])
    def _():
        y = jnp.dot(act_ref[...], wd_bf[...],
                    preferred_element_type=jnp.float32)
        yb = y.astype(jnp.bfloat16).reshape(2 * BM, P)
        ys_ref[...] = pltpu.bitcast(yb, jnp.int32)


def _moe(pref, xs, gup, wd):
    m1_spec = pltpu.PrefetchScalarGridSpec(
        num_scalar_prefetch=1,
        grid=(NB,),
        in_specs=[
            pl.BlockSpec((BM, P), lambda m, p: (m, 0)),
            pl.BlockSpec((1, D, 2 * I), lambda m, p: (p[m], 0, 0),
                         pipeline_mode=_WSPEC),
        ],
        out_specs=pl.BlockSpec((BM, I), lambda m, p: (m, 0)),
        scratch_shapes=[pltpu.VMEM((D, 2 * I), jnp.bfloat16)],
    )
    act = pl.pallas_call(
        _m1_body,
        grid_spec=m1_spec,
        out_shape=jax.ShapeDtypeStruct((CAP, I), jnp.bfloat16),
        compiler_params=pltpu.CompilerParams(
            vmem_limit_bytes=63 * 1024 * 1024),
    )(pref, xs, gup)
    m2_spec = pltpu.PrefetchScalarGridSpec(
        num_scalar_prefetch=1,
        grid=(NB,),
        in_specs=[
            pl.BlockSpec((BM, I), lambda m, p: (m, 0)),
            pl.BlockSpec((1, I, D), lambda m, p: (p[m], 0, 0),
                         pipeline_mode=_WSPEC),
        ],
        out_specs=pl.BlockSpec((BM, P), lambda m, p: (m, 0)),
        scratch_shapes=[pltpu.VMEM((I, D), jnp.bfloat16)],
    )
    return pl.pallas_call(
        _m2_body,
        grid_spec=m2_spec,
        out_shape=jax.ShapeDtypeStruct((CAP, P), jnp.int32),
        compiler_params=pltpu.CompilerParams(
            vmem_limit_bytes=63 * 1024 * 1024),
    )(pref, act, wd)


def _sc_gather(ys, dest2d):
    mesh = plsc.VectorSubcoreMesh(
        core_axis_name="c", subcore_axis_name="s", num_cores=2,
        num_subcores=16)

    @functools.partial(
        pl.kernel,
        out_type=jax.ShapeDtypeStruct((A, P), jnp.int32),
        mesh=mesh,
        scratch_types=[
            pltpu.VMEM((NCH, CH), jnp.int32),
            pltpu.VMEM((CH, P), jnp.int32),
            pltpu.VMEM((CH, P), jnp.int32),
            pltpu.SemaphoreType.DMA,
            pltpu.SemaphoreType.DMA,
            pltpu.SemaphoreType.DMA,
            pltpu.SemaphoreType.DMA,
        ],
    )
    def k(ys_hbm, dest_hbm, yp_hbm, dest_v, rows_a, rows_b,
          gs_a, gs_b, os_a, os_b):
        wid = lax.axis_index("s") * 2 + lax.axis_index("c")
        pltpu.sync_copy(dest_hbm.at[pl.ds(wid * NCH, NCH)], dest_v)
        rows = (rows_a, rows_b)
        gsem = (gs_a, gs_b)
        osem = (os_a, os_b)
        gd = [None, None]
        od = [None, None]
        gd[0] = pltpu.async_copy(ys_hbm.at[dest_v.at[0]], rows[0], gsem[0])
        for c in range(NCH):
            b = c % 2
            if od[1 - b] is not None:
                od[1 - b].wait()
            if c + 1 < NCH:
                gd[1 - b] = pltpu.async_copy(
                    ys_hbm.at[dest_v.at[c + 1]], rows[1 - b], gsem[1 - b])
            gd[b].wait()
            base = wid * (A // NW) + c * CH
            od[b] = pltpu.async_copy(rows[b], yp_hbm.at[pl.ds(base, CH)],
                                     osem[b])
        od[(NCH - 1) % 2].wait()

    return k(ys, dest2d)


def _combine_body(yp_ref, w_ref, o_ref):
    bt = w_ref.shape[0]
    r4 = pltpu.bitcast(yp_ref[...], jnp.bfloat16).reshape(bt, 2, 2, P)
    w = w_ref[...]
    w0 = w[:, 0:1]
    w1 = w[:, 1:2]
    out_lo = r4[:, 0, 0, :] * w0 + r4[:, 1, 0, :] * w1
    out_hi = r4[:, 0, 1, :] * w0 + r4[:, 1, 1, :] * w1
    o_ref[...] = jnp.concatenate([out_lo, out_hi], axis=-1)


def _combine(yp, w):
    bt = 512
    return pl.pallas_call(
        _combine_body,
        grid=(T // bt,),
        in_specs=[
            pl.BlockSpec((2 * bt, P), lambda i: (i, 0)),
            pl.BlockSpec((bt, 2), lambda i: (i, 0)),
        ],
        out_specs=pl.BlockSpec((bt, D), lambda i: (i, 0)),
        out_shape=jax.ShapeDtypeStruct((T, D), jnp.float32),
    )(yp, w)


def kernel(hidden_states, gate_weight, gate_up_weights, down_weights):
    x = hidden_states.reshape(-1, D)
    gw_t = gate_weight.T

    ids, w, xp = _router(x, gw_t)
    dest2, meta = _dispatch(ids.reshape(64, 128))
    dest2d = dest2.reshape(A // CH, CH)
    pref = jnp.concatenate([meta[0, :NB], meta[1, :1]]).astype(jnp.int32)
    tok2d = (jnp.arange(A, dtype=jnp.int32) // 2).reshape(A // CH, CH)

    xs = _sc_scatter(xp, tok2d, dest2d)
    ys = _moe(pref, xs, gate_up_weights, down_weights)
    yp = _sc_gather(ys, dest2d)
    out = _combine(yp, w)
    return out.reshape(hidden_states.shape)
